# baseline (speedup 1.0000x reference)
import jax
import jax.numpy as jnp
from jax import lax
from jax.experimental import pallas as pl
from jax.experimental.pallas import tpu as pltpu
from jax.experimental.pallas import tpu_sc as plsc

ZCH_N = 1000000
DIM = 64
N_BUCKETS = 4
BUCKET_SZ = ZCH_N // N_BUCKETS
NUM_N = 16384

NC = 2
NS = 16
L = 16

IDS_PER_TILE = NUM_N // NS
CHUNK = 128
NCHUNK = IDS_PER_TILE // CHUNK
META_CHUNK = 62496
META_LAST = ZCH_N - 15 * META_CHUNK
NBOUNCE = 4
SUB = META_CHUNK // NBOUNCE
SUB_LAST = META_LAST // NBOUNCE


def _remap16(v):
    h = v.astype(jnp.uint32) * jnp.uint32(2654435761)
    bucket = h & jnp.uint32(N_BUCKETS - 1)
    offset = (h >> jnp.uint32(2)) % jnp.uint32(BUCKET_SZ)
    return (bucket * jnp.uint32(BUCKET_SZ) + offset).astype(jnp.int32)


def _body(values_hbm, table_hbm, meta_hbm,
          emb_hbm, remap_hbm, meta_out_hbm,
          vals_v, idx_f, idx_v, ones_v, rowa_v, rowb_v, bnc_v, meta_sh,
          gsem, wsem):
    cid = lax.axis_index("c")
    sid = lax.axis_index("s")
    base = sid * IDS_PER_TILE

    pltpu.sync_copy(values_hbm.at[pl.ds(base, IDS_PER_TILE)], vals_v)
    for j in range(NCHUNK):
        for i in range(CHUNK // L):
            s = j * CHUNK + i * L
            r = _remap16(vals_v[pl.ds(s, L)])
            idx_f[pl.ds(s, L)] = r
            idx_v[j, pl.ds(i * L, L)] = r

    def _stage_in(off, sub):
        for j in range(NBOUNCE):
            o = off + j * sub
            pltpu.sync_copy(meta_hbm.at[pl.ds(o, sub)], bnc_v.at[pl.ds(0, sub)])
            pltpu.sync_copy(bnc_v.at[pl.ds(0, sub)], meta_sh.at[pl.ds(o, sub)])

    def _stage_out(off, sub):
        for j in range(NBOUNCE):
            o = off + j * sub
            pltpu.sync_copy(meta_sh.at[pl.ds(o, sub)], bnc_v.at[pl.ds(0, sub)])
            pltpu.sync_copy(bnc_v.at[pl.ds(0, sub)], meta_out_hbm.at[pl.ds(o, sub)])

    @pl.when(cid == 0)
    def _meta_phase_a():
        pltpu.sync_copy(idx_f, remap_hbm.at[pl.ds(base, IDS_PER_TILE)])
        for j in range(NCHUNK):
            for i in range(CHUNK // L):
                ones_v[j, pl.ds(i * L, L)] = jnp.full((L,), 1.0, jnp.float32)

        @pl.when(sid < NS - 1)
        def _():
            _stage_in(sid * META_CHUNK, SUB)

        @pl.when(sid == NS - 1)
        def _():
            _stage_in((NS - 1) * META_CHUNK, SUB_LAST)

    @pl.when(cid == 1)
    def _emb_phase():
        bufs = (rowa_v, rowb_v)
        writes = [None, None]
        for j in range(NCHUNK):
            b = j % 2
            if writes[b] is not None:
                writes[b].wait()
            pltpu.async_copy(table_hbm.at[idx_v.at[j]], bufs[b], gsem).wait()
            writes[b] = pltpu.async_copy(
                bufs[b], emb_hbm.at[pl.ds(base + j * CHUNK, CHUNK)], wsem)
        for w in writes:
            w.wait()

    plsc.subcore_barrier()

    @pl.when(cid == 0)
    def _meta_scatter():
        for j in range(NCHUNK):
            pltpu.sync_copy(ones_v.at[j], meta_sh.at[idx_v.at[j]], add=True)

    plsc.subcore_barrier()

    @pl.when(cid == 0)
    def _meta_writeback():
        @pl.when(sid < NS - 1)
        def _():
            _stage_out(sid * META_CHUNK, SUB)

        @pl.when(sid == NS - 1)
        def _():
            _stage_out((NS - 1) * META_CHUNK, SUB_LAST)


def kernel(values, lengths, table, meta):
    del lengths
    k = pl.kernel(
        _body,
        out_type=(
            jax.ShapeDtypeStruct((NUM_N, DIM), jnp.float32),
            jax.ShapeDtypeStruct((NUM_N,), jnp.int32),
            jax.ShapeDtypeStruct((ZCH_N,), jnp.float32),
        ),
        mesh=plsc.VectorSubcoreMesh(
            core_axis_name="c", subcore_axis_name="s",
            num_cores=NC, num_subcores=NS),
        compiler_params=pltpu.CompilerParams(use_tc_tiling_on_sc=False),
        scratch_types=[
            pltpu.VMEM((IDS_PER_TILE,), jnp.int32),
            pltpu.VMEM((IDS_PER_TILE,), jnp.int32),
            pltpu.VMEM((NCHUNK, CHUNK), jnp.int32),
            pltpu.VMEM((NCHUNK, CHUNK), jnp.float32),
            pltpu.VMEM((CHUNK, DIM), jnp.float32),
            pltpu.VMEM((CHUNK, DIM), jnp.float32),
            pltpu.VMEM((SUB_LAST,), jnp.float32),
            pltpu.VMEM_SHARED((ZCH_N,), jnp.float32),
            pltpu.SemaphoreType.DMA,
            pltpu.SemaphoreType.DMA,
        ],
    )
    emb, remapped, meta_new = k(values, table, meta)
    return emb, remapped, meta_new


# speedup vs baseline: 1.3568x; 1.3568x over previous
import jax
import jax.numpy as jnp
from jax import lax
from jax.experimental import pallas as pl
from jax.experimental.pallas import tpu as pltpu
from jax.experimental.pallas import tpu_sc as plsc

ZCH_N = 1000000
DIM = 64
N_BUCKETS = 4
BUCKET_SZ = ZCH_N // N_BUCKETS
NUM_N = 16384

NC = 2
NS = 16
L = 16
ROWS_PER_BLK = 8

IDS_PER_TILE = NUM_N // NS
CHUNK = 128
NCHUNK = IDS_PER_TILE // CHUNK
GCHUNK = 16
NGCHUNK = IDS_PER_TILE // GCHUNK
META_CHUNK = 62496
META_LAST = ZCH_N - 15 * META_CHUNK
NBOUNCE = 4
SUB = META_CHUNK // NBOUNCE
SUB_LAST = META_LAST // NBOUNCE


def _remap16(v):
    h = v.astype(jnp.uint32) * jnp.uint32(2654435761)
    bucket = h & jnp.uint32(N_BUCKETS - 1)
    offset = (h >> jnp.uint32(2)) % jnp.uint32(BUCKET_SZ)
    return (bucket * jnp.uint32(BUCKET_SZ) + offset).astype(jnp.int32)


def _body(values_hbm, table_hbm, meta_hbm,
          emb_hbm, remap_hbm, meta_out_hbm,
          vals_v, idx_f, idx_v, ones_v, blk_v, rows_v, bnc_v,
          meta_sh, gsem, wsem):
    cid = lax.axis_index("c")
    sid = lax.axis_index("s")
    base = sid * IDS_PER_TILE

    pltpu.sync_copy(values_hbm.at[pl.ds(base, IDS_PER_TILE)], vals_v)
    for j in range(NCHUNK):
        for i in range(CHUNK // L):
            s = j * CHUNK + i * L
            r = _remap16(vals_v[pl.ds(s, L)])
            idx_f[pl.ds(s, L)] = r
            idx_v[j, pl.ds(i * L, L)] = r

    def _stage_in(off, sub):
        for j in range(NBOUNCE):
            o = off + j * sub
            pltpu.sync_copy(meta_hbm.at[pl.ds(o, sub)], bnc_v.at[pl.ds(0, sub)])
            pltpu.sync_copy(bnc_v.at[pl.ds(0, sub)], meta_sh.at[pl.ds(o, sub)])

    def _stage_out(off, sub):
        for j in range(NBOUNCE):
            o = off + j * sub
            pltpu.sync_copy(meta_sh.at[pl.ds(o, sub)], bnc_v.at[pl.ds(0, sub)])
            pltpu.sync_copy(bnc_v.at[pl.ds(0, sub)], meta_out_hbm.at[pl.ds(o, sub)])

    @pl.when(cid == 0)
    def _meta_phase_a():
        pltpu.sync_copy(idx_f, remap_hbm.at[pl.ds(base, IDS_PER_TILE)])
        for j in range(NCHUNK):
            for i in range(CHUNK // L):
                ones_v[j, pl.ds(i * L, L)] = jnp.full((L,), 1.0, jnp.float32)

        @pl.when(sid < NS - 1)
        def _():
            _stage_in(sid * META_CHUNK, SUB)

        @pl.when(sid == NS - 1)
        def _():
            _stage_in((NS - 1) * META_CHUNK, SUB_LAST)

    @pl.when(cid == 1)
    def _emb_phase():
        obase = base * DIM

        def _gather_chunk(c, carry):
            cb = c * GCHUNK
            rv = idx_f[pl.ds(cb, L)]
            copies = []
            ks = []
            for i in range(GCHUNK):
                r = rv[i]
                ks.append(r & 7)
                rblk = pl.multiple_of(r - ks[i], ROWS_PER_BLK)
                copies.append(pltpu.async_copy(
                    table_hbm.at[pl.ds(rblk, ROWS_PER_BLK), :],
                    blk_v.at[i], gsem))
            for cp in copies:
                cp.wait()
            for i in range(GCHUNK):
                for j in range(DIM // L):
                    rows_v[pl.ds(i * DIM + j * L, L)] = \
                        blk_v[i, ks[i], pl.ds(j * L, L)]
            pltpu.sync_copy(
                rows_v, emb_hbm.at[pl.ds(obase + cb * DIM, GCHUNK * DIM)])
            return carry

        lax.fori_loop(0, NGCHUNK, _gather_chunk, 0)

    plsc.subcore_barrier()

    @pl.when(cid == 0)
    def _meta_scatter():
        for j in range(NCHUNK):
            pltpu.sync_copy(ones_v.at[j], meta_sh.at[idx_v.at[j]], add=True)

    plsc.subcore_barrier()

    @pl.when(cid == 0)
    def _meta_writeback():
        @pl.when(sid < NS - 1)
        def _():
            _stage_out(sid * META_CHUNK, SUB)

        @pl.when(sid == NS - 1)
        def _():
            _stage_out((NS - 1) * META_CHUNK, SUB_LAST)


def kernel(values, lengths, table, meta):
    del lengths
    k = pl.kernel(
        _body,
        out_type=(
            jax.ShapeDtypeStruct((NUM_N * DIM,), jnp.float32),
            jax.ShapeDtypeStruct((NUM_N,), jnp.int32),
            jax.ShapeDtypeStruct((ZCH_N,), jnp.float32),
        ),
        mesh=plsc.VectorSubcoreMesh(
            core_axis_name="c", subcore_axis_name="s",
            num_cores=NC, num_subcores=NS),
        scratch_types=[
            pltpu.VMEM((IDS_PER_TILE,), jnp.int32),
            pltpu.VMEM((IDS_PER_TILE,), jnp.int32),
            pltpu.VMEM((NCHUNK, CHUNK), jnp.int32),
            pltpu.VMEM((NCHUNK, CHUNK), jnp.float32),
            pltpu.VMEM((GCHUNK, ROWS_PER_BLK, DIM), jnp.float32),
            pltpu.VMEM((GCHUNK * DIM,), jnp.float32),
            pltpu.VMEM((SUB_LAST,), jnp.float32),
            pltpu.VMEM_SHARED((ZCH_N,), jnp.float32),
            pltpu.SemaphoreType.DMA,
            pltpu.SemaphoreType.DMA,
        ],
    )
    emb_flat, remapped, meta_new = k(values, table, meta)
    return emb_flat.reshape(NUM_N, DIM), remapped, meta_new
